# baseline (speedup 1.0000x reference)
import jax
import jax.numpy as jnp
from jax import lax
from jax.experimental import pallas as pl
from jax.experimental.pallas import tpu as pltpu
from jax.experimental.pallas import tpu_sc as plsc

N = 10000
E = 320000
D_IN = 128
D_HID = 512
D_LAT = 30
D_LATP = 32

NC = 2
NS = 16
NW = NC * NS
EPW = E // NW
C = 80
NCHUNK = EPW // C
WCHUNK = 1000
NWTILES = N // WCHUNK
ZROWS = 40

_f32 = jnp.float32


def _mesh():
    return plsc.VectorSubcoreMesh(core_axis_name="c", subcore_axis_name="s")


def _sc_pass1_body(x_hbm, src_hbm, dst_hbm, as_hbm, ad_hbm,
                   u1_hbm, sp_hbm, p_hbm,
                   acc, s_acc, idx_s, idx_d, sidx, asv, adv, pbuf, rows,
                   zb, zb1, sem_i, sem_g, sem_s, sem_sc):
    cid = lax.axis_index("c")
    sid = lax.axis_index("s")
    ebase = (cid * NS + sid) * EPW

    def _zb_zero(r, _):
        for f in range(D_IN // 16):
            zb[r, pl.ds(f * 16, 16)] = jnp.zeros((16,), _f32)
        return ()
    lax.fori_loop(0, ZROWS, _zb_zero, (), unroll=2)

    def _zb1_zero(i, _):
        zb1[pl.ds(i * 16, 16)] = jnp.zeros((16,), _f32)
        return ()
    lax.fori_loop(0, 63, _zb1_zero, (), unroll=4)

    @pl.when(sid < NWTILES)
    def _():
        def _acc_zero(j, _):
            pltpu.sync_copy(zb, acc.at[pl.ds(sid * WCHUNK + j * ZROWS, ZROWS)])
            return ()
        lax.fori_loop(0, WCHUNK // ZROWS, _acc_zero, ())
        pltpu.sync_copy(zb1.at[pl.ds(0, WCHUNK)], s_acc.at[pl.ds(sid * WCHUNK, WCHUNK)])

    plsc.subcore_barrier()

    pltpu.sync_copy(src_hbm.at[pl.ds(ebase, C)], idx_s.at[0])
    pltpu.sync_copy(dst_hbm.at[pl.ds(ebase, C)], idx_d.at[0])
    pltpu.async_copy(as_hbm.at[idx_s.at[0]], asv.at[0], sem_g.at[0])
    pltpu.async_copy(ad_hbm.at[idx_d.at[0]], adv.at[0], sem_g.at[0])
    pltpu.async_copy(x_hbm.at[idx_s.at[0]], rows.at[0], sem_g.at[0])
    pltpu.async_copy(src_hbm.at[pl.ds(ebase + C, C)], idx_s.at[1], sem_i.at[1])
    pltpu.async_copy(dst_hbm.at[pl.ds(ebase + C, C)], idx_d.at[1], sem_i.at[1])

    def _chunk(k, _):
        b3 = lax.rem(k, 3)
        s1 = lax.rem(k + 1, 3)
        b2 = lax.rem(k, 2)
        nb2 = lax.rem(k + 1, 2)
        eb = ebase + k * C

        @pl.when(k >= 2)
        def _():
            pltpu.make_async_copy(rows.at[0], acc.at[pl.ds(0, C)], sem_sc.at[s1]).wait()
            pltpu.make_async_copy(pbuf.at[0], s_acc.at[pl.ds(0, C)], sem_sc.at[s1]).wait()

        @pl.when(k + 1 < NCHUNK)
        def _():
            pltpu.make_async_copy(src_hbm.at[pl.ds(0, C)], idx_s.at[nb2], sem_i.at[nb2]).wait()
            pltpu.make_async_copy(src_hbm.at[pl.ds(0, C)], idx_d.at[nb2], sem_i.at[nb2]).wait()
            pltpu.async_copy(as_hbm.at[idx_s.at[nb2]], asv.at[nb2], sem_g.at[s1])
            pltpu.async_copy(ad_hbm.at[idx_d.at[nb2]], adv.at[nb2], sem_g.at[s1])
            pltpu.async_copy(x_hbm.at[idx_s.at[nb2]], rows.at[s1], sem_g.at[s1])

        pltpu.make_async_copy(as_hbm.at[pl.ds(0, C)], asv.at[0], sem_g.at[b3]).wait()
        pltpu.make_async_copy(ad_hbm.at[pl.ds(0, C)], adv.at[0], sem_g.at[b3]).wait()
        pltpu.make_async_copy(x_hbm.at[pl.ds(0, C)], rows.at[0], sem_g.at[b3]).wait()

        @pl.when(k >= 3)
        def _():
            pltpu.make_async_copy(pbuf.at[0], p_hbm.at[pl.ds(0, C)], sem_s.at[b3]).wait()

        for j in range(C // 16):
            z = asv[b2, pl.ds(j * 16, 16)] + adv[b2, pl.ds(j * 16, 16)]
            sig = 1.0 / (1.0 + jnp.exp(-z))
            pbuf[b3, pl.ds(j * 16, 16)] = jnp.exp(sig)
        pltpu.async_copy(pbuf.at[b3], p_hbm.at[pl.ds(eb, C)], sem_s.at[b3])

        def _scale(j, _):
            pv = pbuf[b3, pl.ds(j * 16, 16)]
            for l in range(16):
                e = j * 16 + l
                p = pv[l]
                for f in range(D_IN // 16):
                    rows[b3, e, pl.ds(f * 16, 16)] = rows[b3, e, pl.ds(f * 16, 16)] * p
            return ()
        lax.fori_loop(0, C // 16, _scale, ())

        for j in range(C // 16):
            sidx[b3, pl.ds(j * 16, 16)] = idx_d[b2, pl.ds(j * 16, 16)]
        pltpu.async_copy(rows.at[b3], acc.at[sidx.at[b3]], sem_sc.at[b3], add=True)
        pltpu.async_copy(pbuf.at[b3], s_acc.at[sidx.at[b3]], sem_sc.at[b3], add=True)

        @pl.when(k + 2 < NCHUNK)
        def _():
            pltpu.async_copy(src_hbm.at[pl.ds(eb + 2 * C, C)], idx_s.at[b2], sem_i.at[b2])
            pltpu.async_copy(dst_hbm.at[pl.ds(eb + 2 * C, C)], idx_d.at[b2], sem_i.at[b2])
        return ()
    lax.fori_loop(0, NCHUNK, _chunk, ())

    for sl in ((NCHUNK - 2) % 3, (NCHUNK - 1) % 3):
        pltpu.make_async_copy(rows.at[0], acc.at[pl.ds(0, C)], sem_sc.at[sl]).wait()
        pltpu.make_async_copy(pbuf.at[0], s_acc.at[pl.ds(0, C)], sem_sc.at[sl]).wait()
    for sl in range(3):
        pltpu.make_async_copy(pbuf.at[0], p_hbm.at[pl.ds(0, C)], sem_s.at[sl]).wait()

    plsc.subcore_barrier()

    @pl.when(sid < NWTILES)
    def _():
        r0 = sid * WCHUNK
        pltpu.sync_copy(acc.at[pl.ds(r0, WCHUNK)], u1_hbm.at[cid, pl.ds(r0, WCHUNK)])
        pltpu.sync_copy(s_acc.at[pl.ds(r0, WCHUNK)], zb1.at[pl.ds(0, WCHUNK)])
        pltpu.sync_copy(zb1.at[pl.ds(0, WCHUNK)], sp_hbm.at[pl.ds(cid * N + r0, WCHUNK)])


def _sc_pass1(x, src, dst, a_src, a_dst):
    return pl.kernel(
        _sc_pass1_body,
        out_type=(
            jax.ShapeDtypeStruct((NC, N, D_IN), _f32),
            jax.ShapeDtypeStruct((NC * N,), _f32),
            jax.ShapeDtypeStruct((E,), _f32),
        ),
        mesh=_mesh(),
        compiler_params=pltpu.CompilerParams(needs_layout_passes=False),
        scratch_types=[
            pltpu.VMEM_SHARED((N, D_IN), _f32),
            pltpu.VMEM_SHARED((N,), _f32),
            pltpu.VMEM((2, C), jnp.int32),
            pltpu.VMEM((2, C), jnp.int32),
            pltpu.VMEM((3, C), jnp.int32),
            pltpu.VMEM((2, C), _f32),
            pltpu.VMEM((2, C), _f32),
            pltpu.VMEM((3, C), _f32),
            pltpu.VMEM((3, C, D_IN), _f32),
            pltpu.VMEM((ZROWS, D_IN), _f32),
            pltpu.VMEM((1008,), _f32),
            pltpu.SemaphoreType.DMA((2,)),
            pltpu.SemaphoreType.DMA((3,)),
            pltpu.SemaphoreType.DMA((3,)),
            pltpu.SemaphoreType.DMA((3,)),
        ],
    )(x, src, dst, a_src, a_dst)


def _sc_pass2_body(x2_hbm, src_hbm, dst_hbm, p_hbm, u3_hbm,
                   acc, idx_s, idx_d, sidx, pbuf, rows, zb, sem_i, sem_g, sem_sc):
    cid = lax.axis_index("c")
    sid = lax.axis_index("s")
    ebase = (cid * NS + sid) * EPW

    def _zb_zero(r, _):
        for f in range(D_LATP // 16):
            zb[r, pl.ds(f * 16, 16)] = jnp.zeros((16,), _f32)
        return ()
    lax.fori_loop(0, ZROWS, _zb_zero, (), unroll=2)

    @pl.when(sid < NWTILES)
    def _():
        def _acc_zero(j, _):
            pltpu.sync_copy(zb, acc.at[pl.ds(sid * WCHUNK + j * ZROWS, ZROWS)])
            return ()
        lax.fori_loop(0, WCHUNK // ZROWS, _acc_zero, ())

    plsc.subcore_barrier()

    pltpu.sync_copy(src_hbm.at[pl.ds(ebase, C)], idx_s.at[0])
    pltpu.sync_copy(dst_hbm.at[pl.ds(ebase, C)], idx_d.at[0])
    pltpu.sync_copy(p_hbm.at[pl.ds(ebase, C)], pbuf.at[0])
    pltpu.async_copy(x2_hbm.at[idx_s.at[0]], rows.at[0], sem_g.at[0])
    pltpu.async_copy(src_hbm.at[pl.ds(ebase + C, C)], idx_s.at[1], sem_i.at[1])
    pltpu.async_copy(dst_hbm.at[pl.ds(ebase + C, C)], idx_d.at[1], sem_i.at[1])
    pltpu.async_copy(p_hbm.at[pl.ds(ebase + C, C)], pbuf.at[1], sem_i.at[1])

    def _chunk(k, _):
        b3 = lax.rem(k, 3)
        s1 = lax.rem(k + 1, 3)
        b2 = lax.rem(k, 2)
        nb2 = lax.rem(k + 1, 2)
        eb = ebase + k * C

        @pl.when(k >= 2)
        def _():
            pltpu.make_async_copy(rows.at[0], acc.at[pl.ds(0, C)], sem_sc.at[s1]).wait()

        @pl.when(k + 1 < NCHUNK)
        def _():
            pltpu.make_async_copy(src_hbm.at[pl.ds(0, C)], idx_s.at[nb2], sem_i.at[nb2]).wait()
            pltpu.make_async_copy(src_hbm.at[pl.ds(0, C)], idx_d.at[nb2], sem_i.at[nb2]).wait()
            pltpu.make_async_copy(p_hbm.at[pl.ds(0, C)], pbuf.at[nb2], sem_i.at[nb2]).wait()
            pltpu.async_copy(x2_hbm.at[idx_s.at[nb2]], rows.at[s1], sem_g.at[s1])

        pltpu.make_async_copy(x2_hbm.at[pl.ds(0, C)], rows.at[0], sem_g.at[b3]).wait()

        def _scale(j, _):
            pv = pbuf[b2, pl.ds(j * 16, 16)]
            for l in range(16):
                e = j * 16 + l
                p = pv[l]
                for f in range(D_LATP // 16):
                    rows[b3, e, pl.ds(f * 16, 16)] = rows[b3, e, pl.ds(f * 16, 16)] * p
            return ()
        lax.fori_loop(0, C // 16, _scale, ())

        for j in range(C // 16):
            sidx[b3, pl.ds(j * 16, 16)] = idx_d[b2, pl.ds(j * 16, 16)]
        pltpu.async_copy(rows.at[b3], acc.at[sidx.at[b3]], sem_sc.at[b3], add=True)

        @pl.when(k + 2 < NCHUNK)
        def _():
            pltpu.async_copy(src_hbm.at[pl.ds(eb + 2 * C, C)], idx_s.at[b2], sem_i.at[b2])
            pltpu.async_copy(dst_hbm.at[pl.ds(eb + 2 * C, C)], idx_d.at[b2], sem_i.at[b2])
            pltpu.async_copy(p_hbm.at[pl.ds(eb + 2 * C, C)], pbuf.at[b2], sem_i.at[b2])
        return ()
    lax.fori_loop(0, NCHUNK, _chunk, ())

    for sl in ((NCHUNK - 2) % 3, (NCHUNK - 1) % 3):
        pltpu.make_async_copy(rows.at[0], acc.at[pl.ds(0, C)], sem_sc.at[sl]).wait()

    plsc.subcore_barrier()

    @pl.when(sid < NWTILES)
    def _():
        r0 = sid * WCHUNK
        pltpu.sync_copy(acc.at[pl.ds(r0, WCHUNK)], u3_hbm.at[cid, pl.ds(r0, WCHUNK)])


def _sc_pass2(x2p, src, dst, p):
    return pl.kernel(
        _sc_pass2_body,
        out_type=jax.ShapeDtypeStruct((NC, N, D_LATP), _f32),
        mesh=_mesh(),
        compiler_params=pltpu.CompilerParams(needs_layout_passes=False,
                                             use_tc_tiling_on_sc=False),
        scratch_types=[
            pltpu.VMEM_SHARED((N, D_LATP), _f32),
            pltpu.VMEM((2, C), jnp.int32),
            pltpu.VMEM((2, C), jnp.int32),
            pltpu.VMEM((3, C), jnp.int32),
            pltpu.VMEM((2, C), _f32),
            pltpu.VMEM((3, C, D_LATP), _f32),
            pltpu.VMEM((ZROWS, D_LATP), _f32),
            pltpu.SemaphoreType.DMA((2,)),
            pltpu.SemaphoreType.DMA((3,)),
            pltpu.SemaphoreType.DMA((3,)),
        ],
    )(x2p, src, dst, p)


_BN = 1000
_GRID = N // _BN


def _tc_att_body(x_ref, w1_ref, att_ref, out_ref):
    v = jnp.dot(w1_ref[...], att_ref[...], preferred_element_type=_f32, precision=lax.Precision.HIGHEST)
    out_ref[...] = jnp.dot(x_ref[...], v, preferred_element_type=_f32, precision=lax.Precision.HIGHEST)


def _tc_att(x, W1, attsd):
    return pl.pallas_call(
        _tc_att_body,
        out_shape=jax.ShapeDtypeStruct((N, 2), _f32),
    )(x, W1, attsd)


def _elu(v):
    return jnp.where(v > 0, v, jnp.exp(jnp.minimum(v, 0.0)) - 1.0)


def _tc_mid_body(u1_ref, sp_ref, w1_ref, w2_ref, out_ref):
    u = u1_ref[0] + u1_ref[1]
    s = jnp.sum(sp_ref[...], axis=1)
    g = u / (s[:, None] + 1e-16)
    x1 = _elu(jnp.dot(g, w1_ref[...], preferred_element_type=_f32, precision=lax.Precision.HIGHEST))
    out_ref[...] = jnp.dot(x1, w2_ref[...], preferred_element_type=_f32, precision=lax.Precision.HIGHEST)


def _tc_mid(u1, sp, W1, W2p):
    return pl.pallas_call(
        _tc_mid_body,
        grid=(_GRID,),
        in_specs=[
            pl.BlockSpec((NC, _BN, D_IN), lambda i: (0, i, 0)),
            pl.BlockSpec((_BN, NC), lambda i: (i, 0)),
            pl.BlockSpec((D_IN, D_HID), lambda i: (0, 0)),
            pl.BlockSpec((D_HID, D_LATP), lambda i: (0, 0)),
        ],
        out_specs=pl.BlockSpec((_BN, D_LATP), lambda i: (i, 0)),
        out_shape=jax.ShapeDtypeStruct((N, D_LATP), _f32),
    )(u1, sp, W1, W2p)


def _tc_post_body(u3_ref, sp_ref, w1_ref, w2_ref, out_ref):
    u = u3_ref[0] + u3_ref[1]
    s = jnp.sum(sp_ref[...], axis=1)
    g = u / (s[:, None] + 1e-16)
    x3 = _elu(lax.dot_general(g, w2_ref[...], (((1,), (1,)), ((), ())),
                              preferred_element_type=_f32, precision=lax.Precision.HIGHEST))
    out_ref[...] = lax.dot_general(x3, w1_ref[...], (((1,), (1,)), ((), ())),
                                   preferred_element_type=_f32, precision=lax.Precision.HIGHEST)


def _tc_post(u3, sp, W1, W2p):
    return pl.pallas_call(
        _tc_post_body,
        grid=(_GRID,),
        in_specs=[
            pl.BlockSpec((NC, _BN, D_LATP), lambda i: (0, i, 0)),
            pl.BlockSpec((_BN, NC), lambda i: (i, 0)),
            pl.BlockSpec((D_IN, D_HID), lambda i: (0, 0)),
            pl.BlockSpec((D_HID, D_LATP), lambda i: (0, 0)),
        ],
        out_specs=pl.BlockSpec((_BN, D_IN), lambda i: (i, 0)),
        out_shape=jax.ShapeDtypeStruct((N, D_IN), _f32),
    )(u3, sp, W1, W2p)


def kernel(x, edge_index, W1, att_src, att_dst, W2):
    src = edge_index[0]
    dst = edge_index[1]
    attsd = jnp.stack([att_src, att_dst], axis=1)
    W2p = jnp.pad(W2, ((0, 0), (0, D_LATP - D_LAT)))

    a2 = _tc_att(x, W1, attsd)
    a_src = a2[:, 0]
    a_dst = a2[:, 1]

    u1, sp_flat, p = _sc_pass1(x, src, dst, a_src, a_dst)
    sp = sp_flat.reshape(NC, N).T

    x2p = _tc_mid(u1, sp, W1, W2p)
    u3 = _sc_pass2(x2p, src, dst, p)
    x4 = _tc_post(u3, sp, W1, W2p)

    return (x2p[:, :D_LAT], x4)


# speedup vs baseline: 1.8310x; 1.8310x over previous
import jax
import jax.numpy as jnp
from jax import lax
from jax.experimental import pallas as pl
from jax.experimental.pallas import tpu as pltpu
from jax.experimental.pallas import tpu_sc as plsc

N = 10000
E = 320000
D_IN = 128
D_HID = 512
D_LAT = 30
D_LATP = 32

NC = 2
NS = 16
NW = NC * NS
EPW = E // NW
C = 80
NCHUNK = EPW // C
WCHUNK = 1000
NWTILES = N // WCHUNK
ZROWS = 40

_f32 = jnp.float32


def _mesh():
    return plsc.VectorSubcoreMesh(core_axis_name="c", subcore_axis_name="s")


def _sc_pass1_body(x_hbm, src_hbm, dst_hbm, as_hbm, ad_hbm,
                   u1_hbm, sp_hbm, p_hbm,
                   acc, s_acc, idx_s, idx_d, sidx, asv, adv, pbuf, rows,
                   zb, zb1, sem_i, sem_g, sem_s, sem_sc):
    cid = lax.axis_index("c")
    sid = lax.axis_index("s")
    ebase = (cid * NS + sid) * EPW

    def _zb_zero(r, _):
        for f in range(D_IN // 16):
            zb[r, pl.ds(f * 16, 16)] = jnp.zeros((16,), _f32)
        return ()
    lax.fori_loop(0, ZROWS, _zb_zero, (), unroll=2)

    def _zb1_zero(i, _):
        zb1[pl.ds(i * 16, 16)] = jnp.zeros((16,), _f32)
        return ()
    lax.fori_loop(0, 63, _zb1_zero, (), unroll=4)

    @pl.when(sid < NWTILES)
    def _():
        def _acc_zero(j, _):
            pltpu.sync_copy(zb, acc.at[pl.ds(sid * WCHUNK + j * ZROWS, ZROWS)])
            return ()
        lax.fori_loop(0, WCHUNK // ZROWS, _acc_zero, ())
        pltpu.sync_copy(zb1.at[pl.ds(0, WCHUNK)], s_acc.at[pl.ds(sid * WCHUNK, WCHUNK)])

    plsc.subcore_barrier()

    pltpu.sync_copy(src_hbm.at[pl.ds(ebase, C)], idx_s.at[0])
    pltpu.sync_copy(dst_hbm.at[pl.ds(ebase, C)], idx_d.at[0])
    pltpu.async_copy(as_hbm.at[idx_s.at[0]], asv.at[0], sem_g.at[0])
    pltpu.async_copy(ad_hbm.at[idx_d.at[0]], adv.at[0], sem_g.at[0])
    pltpu.async_copy(x_hbm.at[idx_s.at[0]], rows.at[0], sem_g.at[0])
    pltpu.async_copy(src_hbm.at[pl.ds(ebase + C, C)], idx_s.at[1], sem_i.at[1])
    pltpu.async_copy(dst_hbm.at[pl.ds(ebase + C, C)], idx_d.at[1], sem_i.at[1])

    def _chunk(k, _):
        b3 = lax.rem(k, 3)
        s1 = lax.rem(k + 1, 3)
        b2 = lax.rem(k, 2)
        nb2 = lax.rem(k + 1, 2)
        eb = ebase + k * C

        @pl.when(k >= 2)
        def _():
            pltpu.make_async_copy(rows.at[0], acc.at[pl.ds(0, C)], sem_sc.at[s1]).wait()
            pltpu.make_async_copy(pbuf.at[0], s_acc.at[pl.ds(0, C)], sem_sc.at[s1]).wait()

        @pl.when(k + 1 < NCHUNK)
        def _():
            pltpu.make_async_copy(src_hbm.at[pl.ds(0, C)], idx_s.at[nb2], sem_i.at[nb2]).wait()
            pltpu.make_async_copy(src_hbm.at[pl.ds(0, C)], idx_d.at[nb2], sem_i.at[nb2]).wait()
            pltpu.async_copy(as_hbm.at[idx_s.at[nb2]], asv.at[nb2], sem_g.at[s1])
            pltpu.async_copy(ad_hbm.at[idx_d.at[nb2]], adv.at[nb2], sem_g.at[s1])
            pltpu.async_copy(x_hbm.at[idx_s.at[nb2]], rows.at[s1], sem_g.at[s1])

        pltpu.make_async_copy(as_hbm.at[pl.ds(0, C)], asv.at[0], sem_g.at[b3]).wait()
        pltpu.make_async_copy(ad_hbm.at[pl.ds(0, C)], adv.at[0], sem_g.at[b3]).wait()
        pltpu.make_async_copy(x_hbm.at[pl.ds(0, C)], rows.at[0], sem_g.at[b3]).wait()

        @pl.when(k >= 3)
        def _():
            pltpu.make_async_copy(pbuf.at[0], p_hbm.at[pl.ds(0, C)], sem_s.at[b3]).wait()

        for j in range(C // 16):
            z = asv[b2, pl.ds(j * 16, 16)] + adv[b2, pl.ds(j * 16, 16)]
            sig = 1.0 / (1.0 + jnp.exp(-z))
            pbuf[b3, pl.ds(j * 16, 16)] = jnp.exp(sig)
        pltpu.async_copy(pbuf.at[b3], p_hbm.at[pl.ds(eb, C)], sem_s.at[b3])

        for j in range(C // 16):
            pv = pbuf[b3, pl.ds(j * 16, 16)]
            for l in range(16):
                e = j * 16 + l
                p = pv[l]
                for f in range(D_IN // 16):
                    rows[b3, e, pl.ds(f * 16, 16)] = rows[b3, e, pl.ds(f * 16, 16)] * p

        for j in range(C // 16):
            sidx[b3, pl.ds(j * 16, 16)] = idx_d[b2, pl.ds(j * 16, 16)]
        pltpu.async_copy(rows.at[b3], acc.at[sidx.at[b3]], sem_sc.at[b3], add=True)
        pltpu.async_copy(pbuf.at[b3], s_acc.at[sidx.at[b3]], sem_sc.at[b3], add=True)

        @pl.when(k + 2 < NCHUNK)
        def _():
            pltpu.async_copy(src_hbm.at[pl.ds(eb + 2 * C, C)], idx_s.at[b2], sem_i.at[b2])
            pltpu.async_copy(dst_hbm.at[pl.ds(eb + 2 * C, C)], idx_d.at[b2], sem_i.at[b2])
        return ()
    lax.fori_loop(0, NCHUNK, _chunk, ())

    for sl in ((NCHUNK - 2) % 3, (NCHUNK - 1) % 3):
        pltpu.make_async_copy(rows.at[0], acc.at[pl.ds(0, C)], sem_sc.at[sl]).wait()
        pltpu.make_async_copy(pbuf.at[0], s_acc.at[pl.ds(0, C)], sem_sc.at[sl]).wait()
    for sl in range(3):
        pltpu.make_async_copy(pbuf.at[0], p_hbm.at[pl.ds(0, C)], sem_s.at[sl]).wait()

    plsc.subcore_barrier()

    @pl.when(sid < NWTILES)
    def _():
        r0 = sid * WCHUNK
        pltpu.sync_copy(acc.at[pl.ds(r0, WCHUNK)], u1_hbm.at[cid, pl.ds(r0, WCHUNK)])
        pltpu.sync_copy(s_acc.at[pl.ds(r0, WCHUNK)], zb1.at[pl.ds(0, WCHUNK)])
        pltpu.sync_copy(zb1.at[pl.ds(0, WCHUNK)], sp_hbm.at[pl.ds(cid * N + r0, WCHUNK)])


def _sc_pass1(x, src, dst, a_src, a_dst):
    return pl.kernel(
        _sc_pass1_body,
        out_type=(
            jax.ShapeDtypeStruct((NC, N, D_IN), _f32),
            jax.ShapeDtypeStruct((NC * N,), _f32),
            jax.ShapeDtypeStruct((E,), _f32),
        ),
        mesh=_mesh(),
        compiler_params=pltpu.CompilerParams(needs_layout_passes=False),
        scratch_types=[
            pltpu.VMEM_SHARED((N, D_IN), _f32),
            pltpu.VMEM_SHARED((N,), _f32),
            pltpu.VMEM((2, C), jnp.int32),
            pltpu.VMEM((2, C), jnp.int32),
            pltpu.VMEM((3, C), jnp.int32),
            pltpu.VMEM((2, C), _f32),
            pltpu.VMEM((2, C), _f32),
            pltpu.VMEM((3, C), _f32),
            pltpu.VMEM((3, C, D_IN), _f32),
            pltpu.VMEM((ZROWS, D_IN), _f32),
            pltpu.VMEM((1008,), _f32),
            pltpu.SemaphoreType.DMA((2,)),
            pltpu.SemaphoreType.DMA((3,)),
            pltpu.SemaphoreType.DMA((3,)),
            pltpu.SemaphoreType.DMA((3,)),
        ],
    )(x, src, dst, a_src, a_dst)


def _sc_pass2_body(x2_hbm, src_hbm, dst_hbm, p_hbm, u3_hbm,
                   acc, idx_s, idx_d, sidx, pbuf, rows, zb, sem_i, sem_g, sem_sc):
    cid = lax.axis_index("c")
    sid = lax.axis_index("s")
    ebase = (cid * NS + sid) * EPW

    def _zb_zero(r, _):
        for f in range(D_LATP // 16):
            zb[r, pl.ds(f * 16, 16)] = jnp.zeros((16,), _f32)
        return ()
    lax.fori_loop(0, ZROWS, _zb_zero, (), unroll=2)

    @pl.when(sid < NWTILES)
    def _():
        def _acc_zero(j, _):
            pltpu.sync_copy(zb, acc.at[pl.ds(sid * WCHUNK + j * ZROWS, ZROWS)])
            return ()
        lax.fori_loop(0, WCHUNK // ZROWS, _acc_zero, ())

    plsc.subcore_barrier()

    pltpu.sync_copy(src_hbm.at[pl.ds(ebase, C)], idx_s.at[0])
    pltpu.sync_copy(dst_hbm.at[pl.ds(ebase, C)], idx_d.at[0])
    pltpu.sync_copy(p_hbm.at[pl.ds(ebase, C)], pbuf.at[0])
    pltpu.async_copy(x2_hbm.at[idx_s.at[0]], rows.at[0], sem_g.at[0])
    pltpu.async_copy(src_hbm.at[pl.ds(ebase + C, C)], idx_s.at[1], sem_i.at[1])
    pltpu.async_copy(dst_hbm.at[pl.ds(ebase + C, C)], idx_d.at[1], sem_i.at[1])
    pltpu.async_copy(p_hbm.at[pl.ds(ebase + C, C)], pbuf.at[1], sem_i.at[1])

    def _chunk(k, _):
        b3 = lax.rem(k, 3)
        s1 = lax.rem(k + 1, 3)
        b2 = lax.rem(k, 2)
        nb2 = lax.rem(k + 1, 2)
        eb = ebase + k * C

        @pl.when(k >= 2)
        def _():
            pltpu.make_async_copy(rows.at[0], acc.at[pl.ds(0, C)], sem_sc.at[s1]).wait()

        @pl.when(k + 1 < NCHUNK)
        def _():
            pltpu.make_async_copy(src_hbm.at[pl.ds(0, C)], idx_s.at[nb2], sem_i.at[nb2]).wait()
            pltpu.make_async_copy(src_hbm.at[pl.ds(0, C)], idx_d.at[nb2], sem_i.at[nb2]).wait()
            pltpu.make_async_copy(p_hbm.at[pl.ds(0, C)], pbuf.at[nb2], sem_i.at[nb2]).wait()
            pltpu.async_copy(x2_hbm.at[idx_s.at[nb2]], rows.at[s1], sem_g.at[s1])

        pltpu.make_async_copy(x2_hbm.at[pl.ds(0, C)], rows.at[0], sem_g.at[b3]).wait()

        for j in range(C // 16):
            pv = pbuf[b2, pl.ds(j * 16, 16)]
            for l in range(16):
                e = j * 16 + l
                p = pv[l]
                for f in range(D_LATP // 16):
                    rows[b3, e, pl.ds(f * 16, 16)] = rows[b3, e, pl.ds(f * 16, 16)] * p

        for j in range(C // 16):
            sidx[b3, pl.ds(j * 16, 16)] = idx_d[b2, pl.ds(j * 16, 16)]
        pltpu.async_copy(rows.at[b3], acc.at[sidx.at[b3]], sem_sc.at[b3], add=True)

        @pl.when(k + 2 < NCHUNK)
        def _():
            pltpu.async_copy(src_hbm.at[pl.ds(eb + 2 * C, C)], idx_s.at[b2], sem_i.at[b2])
            pltpu.async_copy(dst_hbm.at[pl.ds(eb + 2 * C, C)], idx_d.at[b2], sem_i.at[b2])
            pltpu.async_copy(p_hbm.at[pl.ds(eb + 2 * C, C)], pbuf.at[b2], sem_i.at[b2])
        return ()
    lax.fori_loop(0, NCHUNK, _chunk, ())

    for sl in ((NCHUNK - 2) % 3, (NCHUNK - 1) % 3):
        pltpu.make_async_copy(rows.at[0], acc.at[pl.ds(0, C)], sem_sc.at[sl]).wait()

    plsc.subcore_barrier()

    @pl.when(sid < NWTILES)
    def _():
        r0 = sid * WCHUNK
        pltpu.sync_copy(acc.at[pl.ds(r0, WCHUNK)], u3_hbm.at[cid, pl.ds(r0, WCHUNK)])


def _sc_pass2(x2p, src, dst, p):
    return pl.kernel(
        _sc_pass2_body,
        out_type=jax.ShapeDtypeStruct((NC, N, D_LATP), _f32),
        mesh=_mesh(),
        compiler_params=pltpu.CompilerParams(needs_layout_passes=False,
                                             use_tc_tiling_on_sc=False),
        scratch_types=[
            pltpu.VMEM_SHARED((N, D_LATP), _f32),
            pltpu.VMEM((2, C), jnp.int32),
            pltpu.VMEM((2, C), jnp.int32),
            pltpu.VMEM((3, C), jnp.int32),
            pltpu.VMEM((2, C), _f32),
            pltpu.VMEM((3, C, D_LATP), _f32),
            pltpu.VMEM((ZROWS, D_LATP), _f32),
            pltpu.SemaphoreType.DMA((2,)),
            pltpu.SemaphoreType.DMA((3,)),
            pltpu.SemaphoreType.DMA((3,)),
        ],
    )(x2p, src, dst, p)


_BN = 1000
_GRID = N // _BN


def _tc_att_body(x_ref, w1_ref, att_ref, out_ref):
    v = jnp.dot(w1_ref[...], att_ref[...], preferred_element_type=_f32, precision=lax.Precision.HIGHEST)
    out_ref[...] = jnp.dot(x_ref[...], v, preferred_element_type=_f32, precision=lax.Precision.HIGHEST)


def _tc_att(x, W1, attsd):
    return pl.pallas_call(
        _tc_att_body,
        out_shape=jax.ShapeDtypeStruct((N, 2), _f32),
    )(x, W1, attsd)


def _elu(v):
    return jnp.where(v > 0, v, jnp.exp(jnp.minimum(v, 0.0)) - 1.0)


def _tc_mid_body(u1_ref, sp_ref, w1_ref, w2_ref, out_ref):
    u = u1_ref[0] + u1_ref[1]
    s = jnp.sum(sp_ref[...], axis=1)
    g = u / (s[:, None] + 1e-16)
    x1 = _elu(jnp.dot(g, w1_ref[...], preferred_element_type=_f32, precision=lax.Precision.HIGHEST))
    out_ref[...] = jnp.dot(x1, w2_ref[...], preferred_element_type=_f32, precision=lax.Precision.HIGHEST)


def _tc_mid(u1, sp, W1, W2p):
    return pl.pallas_call(
        _tc_mid_body,
        grid=(_GRID,),
        in_specs=[
            pl.BlockSpec((NC, _BN, D_IN), lambda i: (0, i, 0)),
            pl.BlockSpec((_BN, NC), lambda i: (i, 0)),
            pl.BlockSpec((D_IN, D_HID), lambda i: (0, 0)),
            pl.BlockSpec((D_HID, D_LATP), lambda i: (0, 0)),
        ],
        out_specs=pl.BlockSpec((_BN, D_LATP), lambda i: (i, 0)),
        out_shape=jax.ShapeDtypeStruct((N, D_LATP), _f32),
    )(u1, sp, W1, W2p)


def _tc_post_body(u3_ref, sp_ref, w1_ref, w2_ref, out_ref):
    u = u3_ref[0] + u3_ref[1]
    s = jnp.sum(sp_ref[...], axis=1)
    g = u / (s[:, None] + 1e-16)
    x3 = _elu(lax.dot_general(g, w2_ref[...], (((1,), (1,)), ((), ())),
                              preferred_element_type=_f32, precision=lax.Precision.HIGHEST))
    out_ref[...] = lax.dot_general(x3, w1_ref[...], (((1,), (1,)), ((), ())),
                                   preferred_element_type=_f32, precision=lax.Precision.HIGHEST)


def _tc_post(u3, sp, W1, W2p):
    return pl.pallas_call(
        _tc_post_body,
        grid=(_GRID,),
        in_specs=[
            pl.BlockSpec((NC, _BN, D_LATP), lambda i: (0, i, 0)),
            pl.BlockSpec((_BN, NC), lambda i: (i, 0)),
            pl.BlockSpec((D_IN, D_HID), lambda i: (0, 0)),
            pl.BlockSpec((D_HID, D_LATP), lambda i: (0, 0)),
        ],
        out_specs=pl.BlockSpec((_BN, D_IN), lambda i: (i, 0)),
        out_shape=jax.ShapeDtypeStruct((N, D_IN), _f32),
    )(u3, sp, W1, W2p)


def kernel(x, edge_index, W1, att_src, att_dst, W2):
    src = edge_index[0]
    dst = edge_index[1]
    attsd = jnp.stack([att_src, att_dst], axis=1)
    W2p = jnp.pad(W2, ((0, 0), (0, D_LATP - D_LAT)))

    a2 = _tc_att(x, W1, attsd)
    a_src = a2[:, 0]
    a_dst = a2[:, 1]

    u1, sp_flat, p = _sc_pass1(x, src, dst, a_src, a_dst)
    sp = sp_flat.reshape(NC, N).T

    x2p = _tc_mid(u1, sp, W1, W2p)
    u3 = _sc_pass2(x2p, src, dst, p)
    x4 = _tc_post(u3, sp, W1, W2p)

    return (x2p[:, :D_LAT], x4)
